# baseline (speedup 1.0000x reference)
import jax
import jax.numpy as jnp
from jax.experimental import pallas as pl
from jax.experimental.pallas import tpu as pltpu

CTX = 50
DIM = 64
OUT = 100
KV = CTX * DIM


def _tc_body(sent_s, label_s, wtT_h, ltT_h, wv_h, bv_v, wl_v, bl_v,
             out_s, out_l, blocks_v, lblk_v, ecol_v, lcol_v, wv_v, sem, wsem):
    cw = pltpu.make_async_copy(wv_h, wv_v, wsem)
    cw.start()
    lbase = pl.multiple_of(label_s[0] & -128, 128)
    cl = pltpu.make_async_copy(ltT_h.at[:, pl.ds(lbase, 128)], lblk_v, sem)
    cl.start()
    copies = []
    for i in range(CTX):
        base = pl.multiple_of(sent_s[i] & -128, 128)
        c = pltpu.make_async_copy(
            wtT_h.at[:, pl.ds(base, 128)], blocks_v.at[i], sem)
        c.start()
        copies.append(c)
    for i in range(CTX):
        copies[i].wait()
        shift = (-sent_s[i]) & 127
        rolled = pltpu.roll(blocks_v[i], shift, axis=1)
        ecol_v[pl.ds(DIM * i, DIM), :] = rolled[:, 0:1]
    cl.wait()
    lshift = (-label_s[0]) & 127
    lcol_v[...] = pltpu.roll(lblk_v[...], lshift, axis=1)[:, 0:1]

    cw.wait()
    H = KV // 2
    se0 = jax.lax.dot_general(ecol_v[pl.ds(0, H), :], wv_v[:, pl.ds(0, H)],
                              (((0,), (1,)), ((), ())),
                              preferred_element_type=jnp.float32)
    se1 = jax.lax.dot_general(ecol_v[pl.ds(H, H), :], wv_v[:, pl.ds(H, H)],
                              (((0,), (1,)), ((), ())),
                              preferred_element_type=jnp.float32)
    out_s[...] = jnp.maximum(se0 + se1 + bv_v[...], 0.0)
    le = jax.lax.dot_general(lcol_v[...], wl_v[...],
                             (((0,), (0,)), ((), ())),
                             preferred_element_type=jnp.float32)
    out_l[...] = jnp.maximum(le + bl_v[...], 0.0)


_tc_call = pl.pallas_call(
    _tc_body,
    out_shape=(
        jax.ShapeDtypeStruct((1, OUT), jnp.float32),
        jax.ShapeDtypeStruct((1, OUT), jnp.float32),
    ),
    in_specs=[
        pl.BlockSpec(memory_space=pltpu.SMEM),
        pl.BlockSpec(memory_space=pltpu.SMEM),
        pl.BlockSpec(memory_space=pl.ANY),
        pl.BlockSpec(memory_space=pl.ANY),
        pl.BlockSpec(memory_space=pl.ANY),
        pl.BlockSpec(memory_space=pltpu.VMEM),
        pl.BlockSpec(memory_space=pltpu.VMEM),
        pl.BlockSpec(memory_space=pltpu.VMEM),
    ],
    out_specs=(
        pl.BlockSpec(memory_space=pltpu.VMEM),
        pl.BlockSpec(memory_space=pltpu.VMEM),
    ),
    scratch_shapes=[
        pltpu.VMEM((CTX, DIM, 128), jnp.float32),
        pltpu.VMEM((DIM, 128), jnp.float32),
        pltpu.VMEM((KV, 1), jnp.float32),
        pltpu.VMEM((DIM, 1), jnp.float32),
        pltpu.VMEM((OUT, KV), jnp.float32),
        pltpu.SemaphoreType.DMA,
        pltpu.SemaphoreType.DMA,
    ],
    compiler_params=pltpu.CompilerParams(disable_bounds_checks=True),
)


def kernel(sent, label, word_table, label_table, W_vocab, b_vocab, W_label, b_label):
    return _tc_call(
        sent, label, word_table.T, label_table.T,
        W_vocab, b_vocab.reshape(1, OUT), W_label.T, b_label.reshape(1, OUT))


# speedup vs baseline: 1.0152x; 1.0152x over previous
import jax
import jax.numpy as jnp
from jax.experimental import pallas as pl
from jax.experimental.pallas import tpu as pltpu

CTX = 50
DIM = 64
OUT = 100
KV = CTX * DIM


def _tc_body(sent_s, label_s, wtT_h, ltT_h, wv_h, bv_v, wl_v, bl_v,
             out_s, out_l, blocks_v, lblk_v, ecol_v, lcol_v, wv_v, sem, wsem):
    cw = pltpu.make_async_copy(wv_h, wv_v, wsem)
    cw.start()
    lbase = pl.multiple_of(label_s[0] & -128, 128)
    cl = pltpu.make_async_copy(ltT_h.at[:, pl.ds(lbase, 128)], lblk_v, sem)
    cl.start()
    copies = []
    for i in range(CTX):
        base = pl.multiple_of(sent_s[i] & -128, 128)
        c = pltpu.make_async_copy(
            wtT_h.at[:, pl.ds(base, 128)], blocks_v.at[i], sem)
        c.start()
        copies.append(c)
    for i in range(CTX):
        copies[i].wait()
        shift = (-sent_s[i]) & 127
        rolled = pltpu.roll(blocks_v[i], shift, axis=1)
        ecol_v[pl.ds(DIM * i, DIM), :] = rolled[:, 0:1]
    cl.wait()
    lshift = (-label_s[0]) & 127
    lcol_v[...] = pltpu.roll(lblk_v[...], lshift, axis=1)[:, 0:1]

    cw.wait()
    se = jax.lax.dot_general(ecol_v[...], wv_v[...],
                             (((0,), (1,)), ((), ())),
                             preferred_element_type=jnp.float32)
    out_s[...] = jnp.maximum(se + bv_v[...], 0.0)
    le = jax.lax.dot_general(lcol_v[...], wl_v[...],
                             (((0,), (0,)), ((), ())),
                             preferred_element_type=jnp.float32)
    out_l[...] = jnp.maximum(le + bl_v[...], 0.0)


_tc_call = pl.pallas_call(
    _tc_body,
    out_shape=(
        jax.ShapeDtypeStruct((1, OUT), jnp.float32),
        jax.ShapeDtypeStruct((1, OUT), jnp.float32),
    ),
    in_specs=[
        pl.BlockSpec(memory_space=pltpu.SMEM),
        pl.BlockSpec(memory_space=pltpu.SMEM),
        pl.BlockSpec(memory_space=pl.ANY),
        pl.BlockSpec(memory_space=pl.ANY),
        pl.BlockSpec(memory_space=pl.ANY),
        pl.BlockSpec(memory_space=pltpu.VMEM),
        pl.BlockSpec(memory_space=pltpu.VMEM),
        pl.BlockSpec(memory_space=pltpu.VMEM),
    ],
    out_specs=(
        pl.BlockSpec(memory_space=pltpu.VMEM),
        pl.BlockSpec(memory_space=pltpu.VMEM),
    ),
    scratch_shapes=[
        pltpu.VMEM((CTX, DIM, 128), jnp.float32),
        pltpu.VMEM((DIM, 128), jnp.float32),
        pltpu.VMEM((KV, 1), jnp.float32),
        pltpu.VMEM((DIM, 1), jnp.float32),
        pltpu.VMEM((OUT, KV), jnp.float32),
        pltpu.SemaphoreType.DMA,
        pltpu.SemaphoreType.DMA,
    ],
    compiler_params=pltpu.CompilerParams(disable_bounds_checks=True),
)


def kernel(sent, label, word_table, label_table, W_vocab, b_vocab, W_label, b_label):
    return _tc_call(
        sent, label, word_table.T, label_table.T,
        W_vocab, b_vocab.reshape(1, OUT), W_label.T, b_label.reshape(1, OUT))
